# baseline (speedup 1.0000x reference)
import functools

import jax
import jax.numpy as jnp
from jax import lax
from jax.experimental import pallas as pl
from jax.experimental.pallas import tpu as pltpu
from jax.experimental.pallas import tpu_sc as plsc

N_NODES = 10000
NP = 10240
E = 160000
EP = 163840
NTILES = 32
RPT = NP // 16
CH = 128
PER_TILE = EP // NTILES
N_CHUNKS = PER_TILE // CH


def _make_agg(num_convs, D, ch, nbuf, n0=None):
    n_chunks = EP // NTILES // ch
    if n0 is None:
        n0 = n_chunks
    n1 = 2 * n_chunks - n0
    assert n0 % nbuf == 0 and n1 % nbuf == 0, (n0, n1, nbuf)
    mesh = plsc.VectorSubcoreMesh(core_axis_name="c", subcore_axis_name="s")
    out_type = [jax.ShapeDtypeStruct((2, NP, D), jnp.float32)] * num_convs
    scratch = (
        [pltpu.VMEM_SHARED((NP, D), jnp.float32)]
        + [pltpu.VMEM((ch,), jnp.int32)] * nbuf
        + [pltpu.VMEM((ch,), jnp.int32)] * nbuf
        + [pltpu.VMEM((ch, D), jnp.float32)] * nbuf
        + [pltpu.SemaphoreType.DMA] * 3
    )

    @functools.partial(pl.kernel, out_type=out_type, mesh=mesh,
                       scratch_types=scratch,
                       compiler_params=pltpu.CompilerParams(
                           use_tc_tiling_on_sc=False))
    def agg(*refs):
        ins = refs[:1 + 3 * num_convs]
        outs = refs[1 + 3 * num_convs: 1 + 4 * num_convs]
        rest = refs[1 + 4 * num_convs:]
        acc = rest[0]
        srcv = rest[1:1 + nbuf]
        dstv = rest[1 + nbuf:1 + 2 * nbuf]
        rows = rest[1 + 2 * nbuf:1 + 3 * nbuf]
        gsem, ssem, isem = rest[1 + 3 * nbuf:1 + 3 * nbuf + 3]
        zeros = ins[0]
        cid = lax.axis_index("c")
        sid = lax.axis_index("s")
        row0 = pl.multiple_of(sid * RPT, RPT)
        chunk0 = jnp.where(cid == 0, sid * n0, 16 * n0 + sid * n1)
        ngroups = jnp.where(cid == 0, n0 // nbuf, n1 // nbuf)
        for c in range(num_convs):
            table = ins[1 + 3 * c]
            src = ins[2 + 3 * c]
            dst = ins[3 + 3 * c]
            out = outs[c]
            pltpu.sync_copy(zeros.at[pl.ds(row0, RPT)],
                            acc.at[pl.ds(row0, RPT)])
            plsc.subcore_barrier()

            def group(g, carry, table=table, src=src, dst=dst):
                base = g * nbuf
                ids = ([pltpu.async_copy(src.at[chunk0 + base + b],
                                         srcv[b], isem)
                        for b in range(nbuf)]
                       + [pltpu.async_copy(dst.at[chunk0 + base + b],
                                           dstv[b], isem)
                          for b in range(nbuf)])
                for d in ids:
                    d.wait()
                gds = [pltpu.async_copy(table.at[srcv[b]], rows[b], gsem)
                       for b in range(nbuf)]
                for d in gds:
                    d.wait()
                sds = [pltpu.async_copy(rows[b], acc.at[dstv[b]],
                                        ssem, add=True)
                       for b in range(nbuf)]
                for d in sds:
                    d.wait()
                return carry

            lax.fori_loop(0, ngroups, group, 0)
            plsc.subcore_barrier()
            pltpu.sync_copy(acc.at[pl.ds(row0, RPT)],
                            out.at[cid, pl.ds(row0, RPT)])
            plsc.subcore_barrier()

    return agg


def _rs_of_deg16(p):
    d = jnp.max(p[0] + p[1], axis=1, keepdims=True)
    return lax.rsqrt(jnp.maximum(d, 1.0))


BLK = 2048


def _scale_call(pdof, pdobb, pdoby, xu, xi):
    def body(pf, pbb, pby, xu_r, xi_r, tuf, tib, tub):
        ones = jnp.ones((BLK, 16), jnp.float32)
        xuv = xu_r[...]
        xiv = xi_r[...]
        tuf[...] = jnp.concatenate([xuv * _rs_of_deg16(pf[...]), ones], axis=1)
        tib[...] = jnp.concatenate([xiv * _rs_of_deg16(pbb[...]), ones], axis=1)
        tub[...] = jnp.concatenate([xuv * _rs_of_deg16(pby[...]), ones], axis=1)

    deg_spec = pl.BlockSpec((2, BLK, 16), lambda i: (0, i, 0))
    x_spec = pl.BlockSpec((BLK, 128), lambda i: (i, 0))
    t_spec = pl.BlockSpec((BLK, 144), lambda i: (i, 0))
    return pl.pallas_call(
        body, grid=(NP // BLK,),
        in_specs=[deg_spec] * 3 + [x_spec] * 2,
        out_specs=[t_spec] * 3,
        out_shape=[jax.ShapeDtypeStruct((NP, 144), jnp.float32)] * 3,
    )(pdof, pdobb, pdoby, xu, xi)


BLKM = 1024


def _mm_call(p1f, p1bb, p1by, pdof, pdobb,
             W1f, W1bb, W1by, b1f, b1bb, b1by, W2f, W2bb):
    def body(p1f_r, p1bb_r, p1by_r, pdof_r, pdobb_r,
             w1f, w1bb, w1by, bf, bbb, bby, w2f, w2bb,
             hu, hi, tf, tbb, rdf, rdbb):
        def split(pref):
            p = pref[...]
            a = p[0][:, :128] + p[1][:, :128]
            d = jnp.max(p[0][:, 128:] + p[1][:, 128:], axis=1, keepdims=True)
            return a, lax.rsqrt(jnp.maximum(d, 1.0))

        af, rf = split(p1f_r)
        abb, rbb = split(p1bb_r)
        aby, rby = split(p1by_r)
        bias_u = (bf[...] + bbb[...]).reshape(1, 256)
        u = (jnp.dot(af * rf, w1f[...], preferred_element_type=jnp.float32)
             + jnp.dot(abb * rbb, w1bb[...], preferred_element_type=jnp.float32)
             + bias_u)
        huv = jnp.maximum(u, 0.0)
        hu[...] = huv
        iv = (jnp.dot(aby * rby, w1by[...], preferred_element_type=jnp.float32)
              + bby[...].reshape(1, 256))
        hiv = jnp.maximum(iv, 0.0)
        hi[...] = hiv
        rdof = _rs_of_deg16(pdof_r[...])
        rdobb = _rs_of_deg16(pdobb_r[...])
        tf[...] = jnp.dot(huv * rdof, w2f[...],
                          preferred_element_type=jnp.float32)
        tbb[...] = jnp.dot(hiv * rdobb, w2bb[...],
                           preferred_element_type=jnp.float32)
        rdf[...] = jnp.broadcast_to(rf, (BLKM, 16))
        rdbb[...] = jnp.broadcast_to(rbb, (BLKM, 16))

    p_spec = pl.BlockSpec((2, BLKM, 144), lambda i: (0, i, 0))
    deg_spec = pl.BlockSpec((2, BLKM, 16), lambda i: (0, i, 0))
    w1_spec = pl.BlockSpec((128, 256), lambda i: (0, 0))
    w2_spec = pl.BlockSpec((256, 128), lambda i: (0, 0))
    b_spec = pl.BlockSpec((1, 1, 256), lambda i: (0, 0, 0))
    h_spec = pl.BlockSpec((BLKM, 256), lambda i: (i, 0))
    t_spec = pl.BlockSpec((BLKM, 128), lambda i: (i, 0))
    r_spec = pl.BlockSpec((BLKM, 16), lambda i: (i, 0))
    return pl.pallas_call(
        body, grid=(NP // BLKM,),
        in_specs=[p_spec] * 3 + [deg_spec] * 2 + [w1_spec] * 3
                 + [b_spec] * 3 + [w2_spec] * 2,
        out_specs=[h_spec, h_spec, t_spec, t_spec, r_spec, r_spec],
        out_shape=[
            jax.ShapeDtypeStruct((NP, 256), jnp.float32),
            jax.ShapeDtypeStruct((NP, 256), jnp.float32),
            jax.ShapeDtypeStruct((NP, 128), jnp.float32),
            jax.ShapeDtypeStruct((NP, 128), jnp.float32),
            jax.ShapeDtypeStruct((NP, 16), jnp.float32),
            jax.ShapeDtypeStruct((NP, 16), jnp.float32),
        ],
    )(p1f, p1bb, p1by, pdof, pdobb, W1f, W1bb, W1by, b1f, b1bb, b1by,
      W2f, W2bb)


def _final_call(p2f, p2bb, rdf, rdbb, b2f, b2bb):
    def body(pf, pbb, rf, rbb, bf, bbb, pro):
        pfv = pf[...]
        pbbv = pbb[...]
        o = ((pfv[0] + pfv[1]) * jnp.max(rf[...], axis=1, keepdims=True)
             + (pbbv[0] + pbbv[1]) * jnp.max(rbb[...], axis=1, keepdims=True)
             + (bf[...] + bbb[...]).reshape(1, 128))
        m = jnp.max(o, axis=1, keepdims=True)
        e = jnp.exp(o - m)
        pro[...] = e / jnp.sum(e, axis=1, keepdims=True)

    p_spec = pl.BlockSpec((2, BLK, 128), lambda i: (0, i, 0))
    r_spec = pl.BlockSpec((BLK, 16), lambda i: (i, 0))
    b_spec = pl.BlockSpec((1, 1, 128), lambda i: (0, 0, 0))
    o_spec = pl.BlockSpec((BLK, 128), lambda i: (i, 0))
    return pl.pallas_call(
        body, grid=(NP // BLK,),
        in_specs=[p_spec, p_spec, r_spec, r_spec, b_spec, b_spec],
        out_specs=o_spec,
        out_shape=jax.ShapeDtypeStruct((NP, 128), jnp.float32),
    )(p2f, p2bb, rdf, rdbb, b2f, b2bb)


def kernel(x_user, x_item, edge_follows, edge_boughtby, edge_buys,
           W1_follows, b1_follows, W1_boughtby, b1_boughtby, W1_buys, b1_buys,
           W2_follows, b2_follows, W2_boughtby, b2_boughtby):
    f32 = jnp.float32
    pad_idx = jnp.full((EP - E,), N_NODES, jnp.int32)

    def pad_e(e):
        return (jnp.concatenate([e[0], pad_idx]),
                jnp.concatenate([e[1], pad_idx]))

    def r(e, ch):
        return e.reshape(EP // ch, ch)

    f0, f1 = pad_e(edge_follows)
    bb0, bb1 = pad_e(edge_boughtby)
    by0, by1 = pad_e(edge_buys)
    xu = jnp.pad(x_user, ((0, NP - N_NODES), (0, 0)))
    xi = jnp.pad(x_item, ((0, NP - N_NODES), (0, 0)))
    ones16 = jnp.ones((NP, 16), f32)
    z16 = jnp.zeros((NP, 16), f32)
    z144 = jnp.zeros((NP, 144), f32)
    z128 = jnp.zeros((NP, 128), f32)

    deg_pass = _make_agg(3, 16, 128, 4, n0=32)
    pdof, pdobb, pdoby = deg_pass(
        z16, ones16, r(f0, 128), r(f0, 128), ones16, r(bb0, 128),
        r(bb0, 128), ones16, r(by0, 128), r(by0, 128))

    tuf, tib, tub = _scale_call(pdof, pdobb, pdoby, xu, xi)

    agg3 = _make_agg(3, 144, 64, 4, n0=44)
    p1f, p1bb, p1by = agg3(
        z144, tuf, r(f0, 64), r(f1, 64), tib, r(bb0, 64), r(bb1, 64),
        tub, r(by0, 64), r(by1, 64))

    hu, hi, tf, tbb, rdf, rdbb = _mm_call(
        p1f, p1bb, p1by, pdof, pdobb,
        W1_follows, W1_boughtby, W1_buys,
        b1_follows.reshape(1, 1, -1), b1_boughtby.reshape(1, 1, -1),
        b1_buys.reshape(1, 1, -1), W2_follows, W2_boughtby)

    agg2 = _make_agg(2, 128, 64, 4, n0=40)
    p2f, p2bb = agg2(z128, tf, r(f0, 64), r(f1, 64),
                     tbb, r(bb0, 64), r(bb1, 64))

    pro = _final_call(p2f, p2bb, rdf, rdbb,
                      b2_follows.reshape(1, 1, -1),
                      b2_boughtby.reshape(1, 1, -1))
    return hu[:N_NODES], hi[:N_NODES], pro[:N_NODES]


# speedup vs baseline: 1.2499x; 1.2499x over previous
import functools

import jax
import jax.numpy as jnp
from jax import lax
from jax.experimental import pallas as pl
from jax.experimental.pallas import tpu as pltpu
from jax.experimental.pallas import tpu_sc as plsc

N_NODES = 10000
NP = 10240
E = 160000
EP = 163840
NTILES = 32
RPT = NP // 16
CH = 128
PER_TILE = EP // NTILES
N_CHUNKS = PER_TILE // CH


def _make_agg(num_convs, D, ch, nbuf, n0=None):
    n_chunks = EP // NTILES // ch
    if n0 is None:
        n0 = n_chunks
    n1 = 2 * n_chunks - n0
    assert n0 % nbuf == 0 and n1 % nbuf == 0, (n0, n1, nbuf)
    mesh = plsc.VectorSubcoreMesh(core_axis_name="c", subcore_axis_name="s")
    out_type = [jax.ShapeDtypeStruct((2, NP, D), jnp.float32)] * num_convs
    scratch = (
        [pltpu.VMEM_SHARED((NP, D), jnp.float32)]
        + [pltpu.VMEM((ch,), jnp.int32)] * nbuf
        + [pltpu.VMEM((ch,), jnp.int32)] * nbuf
        + [pltpu.VMEM((ch, D), jnp.float32)] * nbuf
        + [pltpu.SemaphoreType.DMA] * 3
    )

    @functools.partial(pl.kernel, out_type=out_type, mesh=mesh,
                       scratch_types=scratch,
                       compiler_params=pltpu.CompilerParams(
                           use_tc_tiling_on_sc=False))
    def agg(*refs):
        ins = refs[:1 + 3 * num_convs]
        outs = refs[1 + 3 * num_convs: 1 + 4 * num_convs]
        rest = refs[1 + 4 * num_convs:]
        acc = rest[0]
        srcv = rest[1:1 + nbuf]
        dstv = rest[1 + nbuf:1 + 2 * nbuf]
        rows = rest[1 + 2 * nbuf:1 + 3 * nbuf]
        gsem, ssem, isem = rest[1 + 3 * nbuf:1 + 3 * nbuf + 3]
        zeros = ins[0]
        cid = lax.axis_index("c")
        sid = lax.axis_index("s")
        row0 = pl.multiple_of(sid * RPT, RPT)
        chunk0 = jnp.where(cid == 0, sid * n0, 16 * n0 + sid * n1)
        ngroups = jnp.where(cid == 0, n0 // nbuf, n1 // nbuf)
        for c in range(num_convs):
            table = ins[1 + 3 * c]
            src = ins[2 + 3 * c]
            dst = ins[3 + 3 * c]
            out = outs[c]
            pltpu.sync_copy(zeros.at[pl.ds(row0, RPT)],
                            acc.at[pl.ds(row0, RPT)])
            plsc.subcore_barrier()

            def group(g, carry, table=table, src=src, dst=dst):
                base = g * nbuf
                ids = ([pltpu.async_copy(src.at[chunk0 + base + b],
                                         srcv[b], isem)
                        for b in range(nbuf)]
                       + [pltpu.async_copy(dst.at[chunk0 + base + b],
                                           dstv[b], isem)
                          for b in range(nbuf)])
                for d in ids:
                    d.wait()
                gds = [pltpu.async_copy(table.at[srcv[b]], rows[b], gsem)
                       for b in range(nbuf)]
                for d in gds:
                    d.wait()
                sds = [pltpu.async_copy(rows[b], acc.at[dstv[b]],
                                        ssem, add=True)
                       for b in range(nbuf)]
                for d in sds:
                    d.wait()
                return carry

            lax.fori_loop(0, ngroups, group, 0)
            plsc.subcore_barrier()
            pltpu.sync_copy(acc.at[pl.ds(row0, RPT)],
                            out.at[cid, pl.ds(row0, RPT)])
            plsc.subcore_barrier()

    return agg


def _rs_of_deg16(p):
    d = jnp.max(p[0] + p[1], axis=1, keepdims=True)
    return lax.rsqrt(jnp.maximum(d, 1.0))


BLK = 2048


def _scale_call(pdof, pdobb, pdoby, xu, xi):
    def body(pf, pbb, pby, xu_r, xi_r, tuf, tib, tub):
        ones = jnp.ones((BLK, 16), jnp.float32)
        xuv = xu_r[...]
        xiv = xi_r[...]
        tuf[...] = jnp.concatenate([xuv * _rs_of_deg16(pf[...]), ones], axis=1)
        tib[...] = jnp.concatenate([xiv * _rs_of_deg16(pbb[...]), ones], axis=1)
        tub[...] = jnp.concatenate([xuv * _rs_of_deg16(pby[...]), ones], axis=1)

    deg_spec = pl.BlockSpec((2, BLK, 16), lambda i: (0, i, 0))
    x_spec = pl.BlockSpec((BLK, 128), lambda i: (i, 0))
    t_spec = pl.BlockSpec((BLK, 144), lambda i: (i, 0))
    return pl.pallas_call(
        body, grid=(NP // BLK,),
        in_specs=[deg_spec] * 3 + [x_spec] * 2,
        out_specs=[t_spec] * 3,
        out_shape=[jax.ShapeDtypeStruct((NP, 144), jnp.float32)] * 3,
    )(pdof, pdobb, pdoby, xu, xi)


BLKM = 1024


def _mm_call(p1f, p1bb, p1by, pdof, pdobb,
             W1f, W1bb, W1by, b1f, b1bb, b1by, W2f, W2bb):
    def body(p1f_r, p1bb_r, p1by_r, pdof_r, pdobb_r,
             w1f, w1bb, w1by, bf, bbb, bby, w2f, w2bb,
             hu, hi, tf, tbb, rdf, rdbb):
        def split(pref):
            p = pref[...]
            a = p[0][:, :128] + p[1][:, :128]
            d = jnp.max(p[0][:, 128:] + p[1][:, 128:], axis=1, keepdims=True)
            return a, lax.rsqrt(jnp.maximum(d, 1.0))

        af, rf = split(p1f_r)
        abb, rbb = split(p1bb_r)
        aby, rby = split(p1by_r)
        bias_u = (bf[...] + bbb[...]).reshape(1, 256)
        u = (jnp.dot(af * rf, w1f[...], preferred_element_type=jnp.float32)
             + jnp.dot(abb * rbb, w1bb[...], preferred_element_type=jnp.float32)
             + bias_u)
        huv = jnp.maximum(u, 0.0)
        hu[...] = huv
        iv = (jnp.dot(aby * rby, w1by[...], preferred_element_type=jnp.float32)
              + bby[...].reshape(1, 256))
        hiv = jnp.maximum(iv, 0.0)
        hi[...] = hiv
        rdof = _rs_of_deg16(pdof_r[...])
        rdobb = _rs_of_deg16(pdobb_r[...])
        tf[...] = jnp.dot(huv * rdof, w2f[...],
                          preferred_element_type=jnp.float32)
        tbb[...] = jnp.dot(hiv * rdobb, w2bb[...],
                           preferred_element_type=jnp.float32)
        rdf[...] = jnp.broadcast_to(rf, (BLKM, 16))
        rdbb[...] = jnp.broadcast_to(rbb, (BLKM, 16))

    p_spec = pl.BlockSpec((2, BLKM, 144), lambda i: (0, i, 0))
    deg_spec = pl.BlockSpec((2, BLKM, 16), lambda i: (0, i, 0))
    w1_spec = pl.BlockSpec((128, 256), lambda i: (0, 0))
    w2_spec = pl.BlockSpec((256, 128), lambda i: (0, 0))
    b_spec = pl.BlockSpec((1, 1, 256), lambda i: (0, 0, 0))
    h_spec = pl.BlockSpec((BLKM, 256), lambda i: (i, 0))
    t_spec = pl.BlockSpec((BLKM, 128), lambda i: (i, 0))
    r_spec = pl.BlockSpec((BLKM, 16), lambda i: (i, 0))
    return pl.pallas_call(
        body, grid=(NP // BLKM,),
        in_specs=[p_spec] * 3 + [deg_spec] * 2 + [w1_spec] * 3
                 + [b_spec] * 3 + [w2_spec] * 2,
        out_specs=[h_spec, h_spec, t_spec, t_spec, r_spec, r_spec],
        out_shape=[
            jax.ShapeDtypeStruct((NP, 256), jnp.float32),
            jax.ShapeDtypeStruct((NP, 256), jnp.float32),
            jax.ShapeDtypeStruct((NP, 128), jnp.float32),
            jax.ShapeDtypeStruct((NP, 128), jnp.float32),
            jax.ShapeDtypeStruct((NP, 16), jnp.float32),
            jax.ShapeDtypeStruct((NP, 16), jnp.float32),
        ],
    )(p1f, p1bb, p1by, pdof, pdobb, W1f, W1bb, W1by, b1f, b1bb, b1by,
      W2f, W2bb)


def _final_call(p2f, p2bb, rdf, rdbb, b2f, b2bb):
    def body(pf, pbb, rf, rbb, bf, bbb, pro):
        pfv = pf[...]
        pbbv = pbb[...]
        o = ((pfv[0] + pfv[1]) * jnp.max(rf[...], axis=1, keepdims=True)
             + (pbbv[0] + pbbv[1]) * jnp.max(rbb[...], axis=1, keepdims=True)
             + (bf[...] + bbb[...]).reshape(1, 128))
        m = jnp.max(o, axis=1, keepdims=True)
        e = jnp.exp(o - m)
        pro[...] = e / jnp.sum(e, axis=1, keepdims=True)

    p_spec = pl.BlockSpec((2, BLK, 128), lambda i: (0, i, 0))
    r_spec = pl.BlockSpec((BLK, 16), lambda i: (i, 0))
    b_spec = pl.BlockSpec((1, 1, 128), lambda i: (0, 0, 0))
    o_spec = pl.BlockSpec((BLK, 128), lambda i: (i, 0))
    return pl.pallas_call(
        body, grid=(NP // BLK,),
        in_specs=[p_spec, p_spec, r_spec, r_spec, b_spec, b_spec],
        out_specs=o_spec,
        out_shape=jax.ShapeDtypeStruct((NP, 128), jnp.float32),
    )(p2f, p2bb, rdf, rdbb, b2f, b2bb)


def kernel(x_user, x_item, edge_follows, edge_boughtby, edge_buys,
           W1_follows, b1_follows, W1_boughtby, b1_boughtby, W1_buys, b1_buys,
           W2_follows, b2_follows, W2_boughtby, b2_boughtby):
    f32 = jnp.float32
    pad_idx = jnp.full((EP - E,), N_NODES, jnp.int32)

    def pad_e(e):
        return (jnp.concatenate([e[0], pad_idx]),
                jnp.concatenate([e[1], pad_idx]))

    def r(e, ch):
        return e.reshape(EP // ch, ch)

    f0, f1 = pad_e(edge_follows)
    bb0, bb1 = pad_e(edge_boughtby)
    by0, by1 = pad_e(edge_buys)
    xu = jnp.pad(x_user, ((0, NP - N_NODES), (0, 0)))
    xi = jnp.pad(x_item, ((0, NP - N_NODES), (0, 0)))
    ones16 = jnp.ones((NP, 16), f32)
    z16 = jnp.zeros((NP, 16), f32)
    z144 = jnp.zeros((NP, 144), f32)
    z128 = jnp.zeros((NP, 128), f32)

    deg_pass = _make_agg(3, 16, 128, 4, n0=48)
    pdof, pdobb, pdoby = deg_pass(
        z16, ones16, r(f0, 128), r(f0, 128), ones16, r(bb0, 128),
        r(bb0, 128), ones16, r(by0, 128), r(by0, 128))

    tuf, tib, tub = _scale_call(pdof, pdobb, pdoby, xu, xi)

    agg3 = _make_agg(3, 144, 64, 4, n0=116)
    p1f, p1bb, p1by = agg3(
        z144, tuf, r(f0, 64), r(f1, 64), tib, r(bb0, 64), r(bb1, 64),
        tub, r(by0, 64), r(by1, 64))

    hu, hi, tf, tbb, rdf, rdbb = _mm_call(
        p1f, p1bb, p1by, pdof, pdobb,
        W1_follows, W1_boughtby, W1_buys,
        b1_follows.reshape(1, 1, -1), b1_boughtby.reshape(1, 1, -1),
        b1_buys.reshape(1, 1, -1), W2_follows, W2_boughtby)

    agg2 = _make_agg(2, 128, 64, 4, n0=120)
    p2f, p2bb = agg2(z128, tf, r(f0, 64), r(f1, 64),
                     tbb, r(bb0, 64), r(bb1, 64))

    pro = _final_call(p2f, p2bb, rdf, rdbb,
                      b2_follows.reshape(1, 1, -1),
                      b2_boughtby.reshape(1, 1, -1))
    return hu[:N_NODES], hi[:N_NODES], pro[:N_NODES]


# speedup vs baseline: 1.7194x; 1.3756x over previous
import functools

import jax
import jax.numpy as jnp
from jax import lax
from jax.experimental import pallas as pl
from jax.experimental.pallas import tpu as pltpu
from jax.experimental.pallas import tpu_sc as plsc

N_NODES = 10000
NP = 10240
E = 160000
EP = 163840
NTILES = 32
RPT = NP // 16
CH = 128
PER_TILE = EP // NTILES
N_CHUNKS = PER_TILE // CH


def _make_agg(num_convs, D, ch, nbuf, n0=None, nsplit=1):
    dh = D // nsplit
    n_chunks = EP // NTILES // ch
    if n0 is None:
        n0 = n_chunks
    n1 = 2 * n_chunks - n0
    assert n0 % nbuf == 0 and n1 % nbuf == 0, (n0, n1, nbuf)
    npc = nsplit + 2
    mesh = plsc.VectorSubcoreMesh(core_axis_name="c", subcore_axis_name="s")
    out_type = [jax.ShapeDtypeStruct((2, NP, dh), jnp.float32)
                ] * (num_convs * nsplit)
    scratch = (
        [pltpu.VMEM_SHARED((NP, dh), jnp.float32)]
        + [pltpu.VMEM_SHARED((NP, dh), jnp.float32)]
        + [pltpu.VMEM((ch,), jnp.int32)] * nbuf
        + [pltpu.VMEM((ch,), jnp.int32)] * nbuf
        + [pltpu.VMEM((ch, dh), jnp.float32)] * nbuf
        + [pltpu.SemaphoreType.DMA] * 3
    )

    @functools.partial(pl.kernel, out_type=out_type, mesh=mesh,
                       scratch_types=scratch,
                       compiler_params=pltpu.CompilerParams(
                           use_tc_tiling_on_sc=False))
    def agg(*refs):
        ins = refs[:1 + npc * num_convs]
        outs = refs[1 + npc * num_convs:
                    1 + npc * num_convs + num_convs * nsplit]
        rest = refs[1 + npc * num_convs + num_convs * nsplit:]
        tab, acc = rest[0], rest[1]
        srcv = rest[2:2 + nbuf]
        dstv = rest[2 + nbuf:2 + 2 * nbuf]
        rows = rest[2 + 2 * nbuf:2 + 3 * nbuf]
        gsem, ssem, isem = rest[2 + 3 * nbuf:2 + 3 * nbuf + 3]
        zeros = ins[0]
        cid = lax.axis_index("c")
        sid = lax.axis_index("s")
        row0 = pl.multiple_of(sid * RPT, RPT)
        chunk0 = jnp.where(cid == 0, sid * n0, 16 * n0 + sid * n1)
        ngroups = jnp.where(cid == 0, n0 // nbuf, n1 // nbuf)
        for c in range(num_convs):
            src = ins[1 + npc * c + nsplit]
            dst = ins[1 + npc * c + nsplit + 1]
            for sp in range(nsplit):
                tabh = ins[1 + npc * c + sp]
                out = outs[c * nsplit + sp]
                pltpu.sync_copy(tabh.at[pl.ds(row0, RPT)],
                                tab.at[pl.ds(row0, RPT)])
                pltpu.sync_copy(zeros.at[pl.ds(row0, RPT)],
                                acc.at[pl.ds(row0, RPT)])
                plsc.subcore_barrier()

                def group(g, carry, src=src, dst=dst):
                    base = g * nbuf
                    ids = ([pltpu.async_copy(src.at[chunk0 + base + b],
                                             srcv[b], isem)
                            for b in range(nbuf)]
                           + [pltpu.async_copy(dst.at[chunk0 + base + b],
                                               dstv[b], isem)
                              for b in range(nbuf)])
                    for d in ids:
                        d.wait()
                    gds = [pltpu.async_copy(tab.at[srcv[b]], rows[b], gsem)
                           for b in range(nbuf)]
                    for d in gds:
                        d.wait()
                    sds = [pltpu.async_copy(rows[b], acc.at[dstv[b]],
                                            ssem, add=True)
                           for b in range(nbuf)]
                    for d in sds:
                        d.wait()
                    return carry

                lax.fori_loop(0, ngroups, group, 0)
                plsc.subcore_barrier()
                pltpu.sync_copy(acc.at[pl.ds(row0, RPT)],
                                out.at[cid, pl.ds(row0, RPT)])
                plsc.subcore_barrier()

    return agg


def _rs_of_deg16(p):
    d = jnp.max(p[0] + p[1], axis=1, keepdims=True)
    return lax.rsqrt(jnp.maximum(d, 1.0))


BLK = 2048


def _scale_call(pdof, pdobb, pdoby, xu, xi):
    def body(pf, pbb, pby, xu_r, xi_r, tuf_lo, tuf_hi, tib_lo, tib_hi,
             tub_lo, tub_hi):
        ones = jnp.ones((BLK, 16), jnp.float32)
        xuv = xu_r[...]
        xiv = xi_r[...]
        for x, p, lo, hi in ((xuv, pf, tuf_lo, tuf_hi),
                             (xiv, pbb, tib_lo, tib_hi),
                             (xuv, pby, tub_lo, tub_hi)):
            xr = x * _rs_of_deg16(p[...])
            lo[...] = xr[:, :72]
            hi[...] = jnp.concatenate([xr[:, 72:], ones], axis=1)

    deg_spec = pl.BlockSpec((2, BLK, 16), lambda i: (0, i, 0))
    x_spec = pl.BlockSpec((BLK, 128), lambda i: (i, 0))
    t_spec = pl.BlockSpec((BLK, 72), lambda i: (i, 0))
    return pl.pallas_call(
        body, grid=(NP // BLK,),
        in_specs=[deg_spec] * 3 + [x_spec] * 2,
        out_specs=[t_spec] * 6,
        out_shape=[jax.ShapeDtypeStruct((NP, 72), jnp.float32)] * 6,
    )(pdof, pdobb, pdoby, xu, xi)


BLKM = 1024


def _mm_call(p1f_lo, p1f_hi, p1bb_lo, p1bb_hi, p1by_lo, p1by_hi,
             pdof, pdobb,
             W1f, W1bb, W1by, b1f, b1bb, b1by, W2f, W2bb):
    def body(p1f_lo, p1f_hi, p1bb_lo, p1bb_hi, p1by_lo, p1by_hi,
             pdof_r, pdobb_r,
             w1f, w1bb, w1by, bf, bbb, bby, w2f, w2bb,
             hu, hi, tf_lo, tf_hi, tbb_lo, tbb_hi, rdf, rdbb):
        def split(loref, hiref):
            lo = loref[...]
            hi = hiref[...]
            hs = hi[0] + hi[1]
            a = jnp.concatenate([lo[0] + lo[1], hs[:, :56]], axis=1)
            d = jnp.max(hs[:, 56:], axis=1, keepdims=True)
            return a, lax.rsqrt(jnp.maximum(d, 1.0))

        af, rf = split(p1f_lo, p1f_hi)
        abb, rbb = split(p1bb_lo, p1bb_hi)
        aby, rby = split(p1by_lo, p1by_hi)
        bias_u = (bf[...] + bbb[...]).reshape(1, 256)
        u = (jnp.dot(af * rf, w1f[...], preferred_element_type=jnp.float32)
             + jnp.dot(abb * rbb, w1bb[...], preferred_element_type=jnp.float32)
             + bias_u)
        huv = jnp.maximum(u, 0.0)
        hu[...] = huv
        iv = (jnp.dot(aby * rby, w1by[...], preferred_element_type=jnp.float32)
              + bby[...].reshape(1, 256))
        hiv = jnp.maximum(iv, 0.0)
        hi[...] = hiv
        rdof = _rs_of_deg16(pdof_r[...])
        rdobb = _rs_of_deg16(pdobb_r[...])
        tfv = jnp.dot(huv * rdof, w2f[...],
                      preferred_element_type=jnp.float32)
        tbbv = jnp.dot(hiv * rdobb, w2bb[...],
                       preferred_element_type=jnp.float32)
        tf_lo[...] = tfv[:, :64]
        tf_hi[...] = tfv[:, 64:]
        tbb_lo[...] = tbbv[:, :64]
        tbb_hi[...] = tbbv[:, 64:]
        rdf[...] = jnp.broadcast_to(rf, (BLKM, 16))
        rdbb[...] = jnp.broadcast_to(rbb, (BLKM, 16))

    p_spec = pl.BlockSpec((2, BLKM, 72), lambda i: (0, i, 0))
    deg_spec = pl.BlockSpec((2, BLKM, 16), lambda i: (0, i, 0))
    w1_spec = pl.BlockSpec((128, 256), lambda i: (0, 0))
    w2_spec = pl.BlockSpec((256, 128), lambda i: (0, 0))
    b_spec = pl.BlockSpec((1, 1, 256), lambda i: (0, 0, 0))
    h_spec = pl.BlockSpec((BLKM, 256), lambda i: (i, 0))
    t_spec = pl.BlockSpec((BLKM, 64), lambda i: (i, 0))
    r_spec = pl.BlockSpec((BLKM, 16), lambda i: (i, 0))
    return pl.pallas_call(
        body, grid=(NP // BLKM,),
        in_specs=[p_spec] * 6 + [deg_spec] * 2 + [w1_spec] * 3
                 + [b_spec] * 3 + [w2_spec] * 2,
        out_specs=[h_spec, h_spec] + [t_spec] * 4 + [r_spec] * 2,
        out_shape=[
            jax.ShapeDtypeStruct((NP, 256), jnp.float32),
            jax.ShapeDtypeStruct((NP, 256), jnp.float32),
            jax.ShapeDtypeStruct((NP, 64), jnp.float32),
            jax.ShapeDtypeStruct((NP, 64), jnp.float32),
            jax.ShapeDtypeStruct((NP, 64), jnp.float32),
            jax.ShapeDtypeStruct((NP, 64), jnp.float32),
            jax.ShapeDtypeStruct((NP, 16), jnp.float32),
            jax.ShapeDtypeStruct((NP, 16), jnp.float32),
        ],
    )(p1f_lo, p1f_hi, p1bb_lo, p1bb_hi, p1by_lo, p1by_hi, pdof, pdobb,
      W1f, W1bb, W1by, b1f, b1bb, b1by, W2f, W2bb)


def _final_call(pf_lo, pf_hi, pbb_lo, pbb_hi, rdf, rdbb, b2f, b2bb):
    def body(pf_lo, pf_hi, pbb_lo, pbb_hi, rf, rbb, bf, bbb, pro):
        def whole(loref, hiref):
            lo = loref[...]
            hi = hiref[...]
            return jnp.concatenate([lo[0] + lo[1], hi[0] + hi[1]], axis=1)

        o = (whole(pf_lo, pf_hi) * jnp.max(rf[...], axis=1, keepdims=True)
             + whole(pbb_lo, pbb_hi) * jnp.max(rbb[...], axis=1, keepdims=True)
             + (bf[...] + bbb[...]).reshape(1, 128))
        m = jnp.max(o, axis=1, keepdims=True)
        e = jnp.exp(o - m)
        pro[...] = e / jnp.sum(e, axis=1, keepdims=True)

    p_spec = pl.BlockSpec((2, BLK, 64), lambda i: (0, i, 0))
    r_spec = pl.BlockSpec((BLK, 16), lambda i: (i, 0))
    b_spec = pl.BlockSpec((1, 1, 128), lambda i: (0, 0, 0))
    o_spec = pl.BlockSpec((BLK, 128), lambda i: (i, 0))
    return pl.pallas_call(
        body, grid=(NP // BLK,),
        in_specs=[p_spec] * 4 + [r_spec, r_spec, b_spec, b_spec],
        out_specs=o_spec,
        out_shape=jax.ShapeDtypeStruct((NP, 128), jnp.float32),
    )(pf_lo, pf_hi, pbb_lo, pbb_hi, rdf, rdbb, b2f, b2bb)


def kernel(x_user, x_item, edge_follows, edge_boughtby, edge_buys,
           W1_follows, b1_follows, W1_boughtby, b1_boughtby, W1_buys, b1_buys,
           W2_follows, b2_follows, W2_boughtby, b2_boughtby):
    f32 = jnp.float32
    pad_idx = jnp.full((EP - E,), N_NODES, jnp.int32)

    def pad_e(e):
        return (jnp.concatenate([e[0], pad_idx]),
                jnp.concatenate([e[1], pad_idx]))

    def r(e, ch):
        return e.reshape(EP // ch, ch)

    f0, f1 = pad_e(edge_follows)
    bb0, bb1 = pad_e(edge_boughtby)
    by0, by1 = pad_e(edge_buys)
    xu = jnp.pad(x_user, ((0, NP - N_NODES), (0, 0)))
    xi = jnp.pad(x_item, ((0, NP - N_NODES), (0, 0)))
    ones16 = jnp.ones((NP, 16), f32)
    z16 = jnp.zeros((NP, 16), f32)
    z72 = jnp.zeros((NP, 72), f32)
    z64 = jnp.zeros((NP, 64), f32)

    deg_pass = _make_agg(3, 16, 64, 16, n0=96)
    pdof, pdobb, pdoby = deg_pass(
        z16, ones16, r(f0, 64), r(f0, 64), ones16, r(bb0, 64),
        r(bb0, 64), ones16, r(by0, 64), r(by0, 64))

    tufl, tufh, tibl, tibh, tubl, tubh = _scale_call(
        pdof, pdobb, pdoby, xu, xi)

    agg3 = _make_agg(3, 144, 32, 8, n0=232, nsplit=2)
    (p1f_lo, p1f_hi, p1bb_lo, p1bb_hi, p1by_lo, p1by_hi) = agg3(
        z72, tufl, tufh, r(f0, 32), r(f1, 32),
        tibl, tibh, r(bb0, 32), r(bb1, 32),
        tubl, tubh, r(by0, 32), r(by1, 32))

    hu, hi, tfl, tfh, tbbl, tbbh, rdf, rdbb = _mm_call(
        p1f_lo, p1f_hi, p1bb_lo, p1bb_hi, p1by_lo, p1by_hi, pdof, pdobb,
        W1_follows, W1_boughtby, W1_buys,
        b1_follows.reshape(1, 1, -1), b1_boughtby.reshape(1, 1, -1),
        b1_buys.reshape(1, 1, -1), W2_follows, W2_boughtby)

    agg2 = _make_agg(2, 128, 32, 8, n0=240, nsplit=2)
    p2f_lo, p2f_hi, p2bb_lo, p2bb_hi = agg2(
        z64, tfl, tfh, r(f0, 32), r(f1, 32),
        tbbl, tbbh, r(bb0, 32), r(bb1, 32))

    pro = _final_call(p2f_lo, p2f_hi, p2bb_lo, p2bb_hi, rdf, rdbb,
                      b2_follows.reshape(1, 1, -1),
                      b2_boughtby.reshape(1, 1, -1))
    return hu[:N_NODES], hi[:N_NODES], pro[:N_NODES]
